# baseline (speedup 1.0000x reference)
import jax
import jax.numpy as jnp
from jax import lax
from jax.experimental import pallas as pl
from jax.experimental.pallas import tpu as pltpu, tpu_sc as plsc

BATCH = 16384
EMBED = 256
CTX = 4
NC = 2
NS = 16
NW = NC * NS
LANES = 16
B_PER_W = BATCH // NW
CHUNK = 16
N_CHUNKS = B_PER_W // CHUNK
NBUF = 4
IDX_W = 128
PID_ROWS = BATCH // IDX_W
CID_ROWS = BATCH * CTX // IDX_W


def _sc_body(para_idx_hbm, ctx_idx_hbm, para_tab_hbm, ctx_tab_hbm, out_hbm,
             pidx_v, cidx_v, prows_v, crows_v, outbuf_v, accbuf_v,
             sem_p, sem_c):
    wid = lax.axis_index("s") * NC + lax.axis_index("c")

    lane_iota = lax.broadcasted_iota(jnp.int32, (LANES,), 0)

    stage_cps = [pltpu.async_copy(
        para_idx_hbm.at[pl.ds(wid * (B_PER_W // IDX_W), B_PER_W // IDX_W)],
        pidx_v, sem_p.at[0])]
    for k in range(CTX):
        stage_cps.append(pltpu.async_copy(
            ctx_idx_hbm.at[pl.ds(k * (BATCH // IDX_W) + wid * (B_PER_W // IDX_W),
                                 B_PER_W // IDX_W)],
            cidx_v.at[pl.ds(k * (B_PER_W // IDX_W), B_PER_W // IDX_W)],
            sem_c.at[0]))
    for cp in stage_cps:
        cp.wait()

    def issue_gather(c, b):
        pltpu.async_copy(
            para_tab_hbm.at[pidx_v.at[c // (IDX_W // CHUNK),
                                      pl.ds((c % (IDX_W // CHUNK)) * CHUNK, CHUNK)]],
            prows_v.at[b], sem_p.at[b])
        rw = IDX_W // CHUNK
        for k in range(CTX):
            pltpu.async_copy(
                ctx_tab_hbm.at[cidx_v.at[k * (B_PER_W // IDX_W) + c // rw,
                                         pl.ds((c % rw) * CHUNK, CHUNK)]],
                crows_v.at[b, pl.ds(k * CHUNK, CHUNK)], sem_c.at[b])

    def wait_gather(b):
        pltpu.make_async_copy(para_tab_hbm.at[pl.ds(0, CHUNK)],
                              prows_v.at[b], sem_p.at[b]).wait()
        pltpu.make_async_copy(ctx_tab_hbm.at[pl.ds(0, CHUNK * CTX)],
                              crows_v.at[b], sem_c.at[b]).wait()

    def compute(c, b):
        def group_body(g, _):
            def row_body(rr, carry):
                r = g * LANES + rr
                acc = jnp.zeros((LANES,), jnp.float32)
                for j in range(EMBED // LANES):
                    sl = pl.ds(j * LANES, LANES)
                    p = prows_v[b, r, sl]
                    s = ((crows_v[b, r, sl] + crows_v[b, CHUNK + r, sl])
                         + (crows_v[b, 2 * CHUNK + r, sl]
                            + crows_v[b, 3 * CHUNK + r, sl]))
                    acc = acc + p * s
                accbuf_v[rr, :] = acc
                return carry

            lax.fori_loop(0, LANES, row_body, 0)
            vec = jnp.zeros((LANES,), jnp.float32)
            for k in range(LANES):
                col = jnp.full((LANES,), k, jnp.int32)
                vec = vec + plsc.load_gather(accbuf_v, [lane_iota, col])
            outbuf_v[pl.ds(c * CHUNK + g * LANES, LANES)] = vec * (1.0 / CTX)
            return 0

        lax.fori_loop(0, CHUNK // LANES, group_body, 0)

    for b in range(NBUF):
        issue_gather(b, b)

    n_groups = N_CHUNKS // NBUF
    assert n_groups * NBUF == N_CHUNKS

    def group_loop(p, carry):
        for b in range(NBUF):
            c = p * NBUF + b
            wait_gather(b)
            compute(c, b)
            nxt = c + NBUF
            @pl.when(nxt < N_CHUNKS)
            def _():
                issue_gather(nxt, b)
        return carry

    lax.fori_loop(0, n_groups, group_loop, 0)
    pltpu.sync_copy(outbuf_v, out_hbm.at[pl.ds(wid * B_PER_W, B_PER_W)])


@jax.jit
def _doc2vec_sc(para_idx, ctx_idx, para_tab, ctx_tab):
    mesh = plsc.VectorSubcoreMesh(core_axis_name="c", subcore_axis_name="s")
    f = pl.kernel(
        _sc_body,
        out_type=jax.ShapeDtypeStruct((BATCH,), jnp.float32),
        mesh=mesh,
        compiler_params=pltpu.CompilerParams(needs_layout_passes=False),
        scratch_types=[
            pltpu.VMEM((B_PER_W // IDX_W, IDX_W), jnp.int32),
            pltpu.VMEM((B_PER_W * CTX // IDX_W, IDX_W), jnp.int32),
            pltpu.VMEM((NBUF, CHUNK, EMBED), jnp.float32),
            pltpu.VMEM((NBUF, CHUNK * CTX, EMBED), jnp.float32),
            pltpu.VMEM((B_PER_W,), jnp.float32),
            pltpu.VMEM((LANES, LANES), jnp.float32),
            pltpu.SemaphoreType.DMA((NBUF,)),
            pltpu.SemaphoreType.DMA((NBUF,)),
        ],
    )
    return f(para_idx, ctx_idx, para_tab, ctx_tab)


def kernel(paragraph, context, paragraph_table, context_table):
    para_idx = paragraph.astype(jnp.int32).T.reshape(PID_ROWS, IDX_W)
    ctx_idx = context.astype(jnp.int32).T.reshape(CID_ROWS, IDX_W)
    return _doc2vec_sc(para_idx, ctx_idx, paragraph_table, context_table)


# speedup vs baseline: 1.0381x; 1.0381x over previous
import jax
import jax.numpy as jnp
from jax import lax
from jax.experimental import pallas as pl
from jax.experimental.pallas import tpu as pltpu, tpu_sc as plsc

BATCH = 16384
EMBED = 256
CTX = 4
NC = 2
NS = 16
NW = NC * NS
LANES = 16
B_PER_W = BATCH // NW
CHUNK = 16
N_CHUNKS = B_PER_W // CHUNK
NBUF = 4
IDX_W = 128
PID_ROWS = BATCH // IDX_W
CID_ROWS = BATCH * CTX // IDX_W


def _sc_body(para_idx_hbm, ctx_idx_hbm, para_tab_hbm, ctx_tab_hbm, out_hbm,
             pidx_v, cidx_v, cidx2_v, prows_v, crows_v, outbuf_v, accbuf_v,
             sem_p, sem_c):
    wid = lax.axis_index("s") * NC + lax.axis_index("c")

    lane_iota = lax.broadcasted_iota(jnp.int32, (LANES,), 0)

    stage_cps = [pltpu.async_copy(
        para_idx_hbm.at[pl.ds(wid * (B_PER_W // IDX_W), B_PER_W // IDX_W)],
        pidx_v, sem_p.at[0])]
    for k in range(CTX):
        stage_cps.append(pltpu.async_copy(
            ctx_idx_hbm.at[pl.ds(k * (BATCH // IDX_W) + wid * (B_PER_W // IDX_W),
                                 B_PER_W // IDX_W)],
            cidx_v.at[pl.ds(k * (B_PER_W // IDX_W), B_PER_W // IDX_W)],
            sem_c.at[0]))
    for cp in stage_cps:
        cp.wait()

    def perm_body(v, carry):
        i = v * LANES + lane_iota
        k = lax.bitwise_and(i, 3)
        bb = lax.shift_right_logical(i, 2)
        row = lax.shift_left(k, 2) + lax.shift_right_logical(bb, 7)
        col = lax.bitwise_and(bb, 127)
        vals = plsc.load_gather(cidx_v, [row, col])
        cidx2_v[v >> 3, pl.ds((v & 7) * LANES, LANES)] = vals
        return carry

    lax.fori_loop(0, B_PER_W * CTX // LANES, perm_body, 0)

    def issue_gather(c, b):
        pltpu.async_copy(
            para_tab_hbm.at[pidx_v.at[c // (IDX_W // CHUNK),
                                      pl.ds((c % (IDX_W // CHUNK)) * CHUNK, CHUNK)]],
            prows_v.at[b], sem_p.at[b])
        cw = IDX_W // (CHUNK * CTX)
        pltpu.async_copy(
            ctx_tab_hbm.at[cidx2_v.at[c // cw,
                                      pl.ds((c % cw) * CHUNK * CTX, CHUNK * CTX)]],
            crows_v.at[b], sem_c.at[b])

    def wait_gather(b):
        pltpu.make_async_copy(para_tab_hbm.at[pl.ds(0, CHUNK)],
                              prows_v.at[b], sem_p.at[b]).wait()
        pltpu.make_async_copy(ctx_tab_hbm.at[pl.ds(0, CHUNK * CTX)],
                              crows_v.at[b], sem_c.at[b]).wait()

    def compute(c, b):
        def group_body(g, _):
            def row_body(rr, carry):
                r = g * LANES + rr
                acc = jnp.zeros((LANES,), jnp.float32)
                for j in range(EMBED // LANES):
                    sl = pl.ds(j * LANES, LANES)
                    p = prows_v[b, r, sl]
                    s = ((crows_v[b, CTX * r, sl] + crows_v[b, CTX * r + 1, sl])
                         + (crows_v[b, CTX * r + 2, sl]
                            + crows_v[b, CTX * r + 3, sl]))
                    acc = acc + p * s
                accbuf_v[rr, :] = acc
                return carry

            lax.fori_loop(0, LANES, row_body, 0)
            vec = jnp.zeros((LANES,), jnp.float32)
            for k in range(LANES):
                col = jnp.full((LANES,), k, jnp.int32)
                vec = vec + plsc.load_gather(accbuf_v, [lane_iota, col])
            outbuf_v[pl.ds(c * CHUNK + g * LANES, LANES)] = vec * (1.0 / CTX)
            return 0

        lax.fori_loop(0, CHUNK // LANES, group_body, 0)

    for b in range(NBUF):
        issue_gather(b, b)

    n_groups = N_CHUNKS // NBUF
    assert n_groups * NBUF == N_CHUNKS

    def group_loop(p, carry):
        for b in range(NBUF):
            c = p * NBUF + b
            wait_gather(b)
            compute(c, b)
            nxt = c + NBUF
            @pl.when(nxt < N_CHUNKS)
            def _():
                issue_gather(nxt, b)
        return carry

    lax.fori_loop(0, n_groups, group_loop, 0)
    pltpu.sync_copy(outbuf_v, out_hbm.at[pl.ds(wid * B_PER_W, B_PER_W)])


@jax.jit
def _doc2vec_sc(para_idx, ctx_idx, para_tab, ctx_tab):
    mesh = plsc.VectorSubcoreMesh(core_axis_name="c", subcore_axis_name="s")
    f = pl.kernel(
        _sc_body,
        out_type=jax.ShapeDtypeStruct((BATCH,), jnp.float32),
        mesh=mesh,
        compiler_params=pltpu.CompilerParams(needs_layout_passes=False),
        scratch_types=[
            pltpu.VMEM((B_PER_W // IDX_W, IDX_W), jnp.int32),
            pltpu.VMEM((B_PER_W * CTX // IDX_W, IDX_W), jnp.int32),
            pltpu.VMEM((B_PER_W * CTX // IDX_W, IDX_W), jnp.int32),
            pltpu.VMEM((NBUF, CHUNK, EMBED), jnp.float32),
            pltpu.VMEM((NBUF, CHUNK * CTX, EMBED), jnp.float32),
            pltpu.VMEM((B_PER_W,), jnp.float32),
            pltpu.VMEM((LANES, LANES), jnp.float32),
            pltpu.SemaphoreType.DMA((NBUF,)),
            pltpu.SemaphoreType.DMA((NBUF,)),
        ],
    )
    return f(para_idx, ctx_idx, para_tab, ctx_tab)


def kernel(paragraph, context, paragraph_table, context_table):
    para_idx = paragraph.astype(jnp.int32).T.reshape(PID_ROWS, IDX_W)
    ctx_idx = context.astype(jnp.int32).T.reshape(CID_ROWS, IDX_W)
    return _doc2vec_sc(para_idx, ctx_idx, paragraph_table, context_table)
